# baseline (speedup 1.0000x reference)
import math

import jax
import jax.numpy as jnp
from jax import lax
from jax.experimental import pallas as pl
from jax.experimental.pallas import tpu as pltpu

_HALVES = (16.0, 32.0, 64.0)
_IMG = 512.0
_THIRD = 1.0 / 3.0
_TWO7 = 2.0 / 7.0


def _body(cls_ref, reg_ref, tb_ref, tbT_ref, out_ref, wxq_scr, wyT_scr, sc_scr):
    f32 = jnp.float32
    tb = tb_ref[0]
    gx1 = tb[:, 0:1]
    gy1 = tb[:, 1:2]
    gx2 = tb[:, 2:3]
    gy2 = tb[:, 3:4]
    tbT = tbT_ref[0]
    gy1r = tbT[1:2, :]
    gy2r = tbT[3:4, :]

    lane = lax.broadcasted_iota(jnp.int32, (1, 128), 1).astype(f32)
    cxr = (lane + 0.5) * 4.0
    subl = lax.broadcasted_iota(jnp.int32, (128, 1), 0).astype(f32)
    cyc = (subl + 0.5) * 4.0

    area_g = (gx2 - gx1) * (gy2 - gy1)
    gt_best = jnp.full((64, 1), -1.0, f32)

    for si, h in enumerate(_HALVES):
        wx = jnp.clip(jnp.minimum(cxr + h, gx2) - jnp.maximum(cxr - h, gx1), 0.0)
        wyr = jnp.clip(jnp.minimum(cxr + h, gy2) - jnp.maximum(cxr - h, gy1), 0.0)
        C = area_g + (4.0 * h * h)
        wxq_scr[si] = wx * (1.0 / C)
        wyT = jnp.clip(jnp.minimum(cyc + h, gy2r) - jnp.maximum(cyc - h, gy1r), 0.0)
        wyT_scr[si] = wyT
        mx = jnp.max(wx, axis=1, keepdims=True)
        my = jnp.max(wyr, axis=1, keepdims=True)
        imax = mx * my
        iou = imax / jnp.maximum(C - imax, 1e-6)
        gt_best = jnp.maximum(gt_best, iou)

    t = gt_best - 1e-3
    thr_lowq = jnp.where(gt_best > 1e-3, t / (1.0 + t), 2.0)
    minthr = jnp.minimum(thr_lowq, _THIRD)
    ones_r = jnp.ones((1, 128), f32)
    sc_scr[0] = minthr * ones_r

    gx1r = tbT[0:1, :]
    gx2r = tbT[2:3, :]
    gcx_r = (gx1r + gx2r) * 0.5
    gcy_r = (gy1r + gy2r) * 0.5
    glw_r = jnp.log(jnp.maximum(gx2r - gx1r, 1e-6))
    glh_r = jnp.log(jnp.maximum(gy2r - gy1r, 1e-6))

    cls_s = jnp.float32(0.0)
    bbox_s = jnp.float32(0.0)
    ns_s = jnp.float32(0.0)

    TR = 32
    for si, h in enumerate(_HALVES):
        for ti in range(128 // TR):
            y0 = ti * TR
            wyT_t = wyT_scr[si, y0:y0 + TR, :]

            def gt_step(g8, st, si=si, wyT_t=wyT_t):
                m, pm, idxf = st
                gb = g8 * 8
                rows = wxq_scr[si, pl.ds(gb, 8), :]
                thr8 = sc_scr[0, pl.ds(gb, 8), :]
                gbf = gb.astype(f32)
                cols = pltpu.roll(wyT_t, -gb, axis=1)
                for j in range(8):
                    q = cols[:, j:j + 1] * rows[j:j + 1, :]
                    win = q > m
                    m = jnp.maximum(m, q)
                    pm = jnp.maximum(pm, q - thr8[j:j + 1, :])
                    idxf = jnp.where(win, gbf + float(j), idxf)
                return m, pm, idxf

            zero = jnp.zeros((TR, 128), f32)
            init = (jnp.full((TR, 128), -1.0, f32),
                    jnp.full((TR, 128), -2.0, f32), zero)
            m, pm, idxf = lax.fori_loop(0, 8, gt_step, init)
            ii = idxf.astype(jnp.int32)
            mcx = jnp.take_along_axis(jnp.broadcast_to(gcx_r, (TR, 128)), ii, axis=1)
            mcy = jnp.take_along_axis(jnp.broadcast_to(gcy_r, (TR, 128)), ii, axis=1)
            mlw = jnp.take_along_axis(jnp.broadcast_to(glw_r, (TR, 128)), ii, axis=1)
            mlh = jnp.take_along_axis(jnp.broadcast_to(glh_r, (TR, 128)), ii, axis=1)

            cyt = cyc[y0:y0 + TR]
            okx = (cxr >= h) & (cxr <= _IMG - h)
            oky = (cyt >= h) & (cyt <= _IMG - h)
            inside = okx & oky
            pos = (pm >= 0.0) & inside
            neg = (m < _TWO7) & (~pos) & inside
            posf = pos.astype(f32)
            lw = posf + neg.astype(f32)

            c = cls_ref[0, si, y0:y0 + TR, :]
            bce = jnp.maximum(c, 0.0) - c * posf + jnp.log1p(jnp.exp(-jnp.abs(c)))
            cls_s = cls_s + jnp.sum(bce * lw)

            aw = 2.0 * h
            dx = (mcx - cxr) / aw
            dy = (mcy - cyt) / aw
            dw = mlw - math.log(aw)
            dh = mlh - math.log(aw)
            r = reg_ref[0, 4 * si:4 * si + 4, y0:y0 + TR, :]
            l1 = (jnp.abs(r[0] - dx) + jnp.abs(r[1] - dy)
                  + jnp.abs(r[2] - dw) + jnp.abs(r[3] - dh))
            bbox_s = bbox_s + jnp.sum(l1 * posf)
            ns_s = ns_s + jnp.sum(lw)

    li = lax.broadcasted_iota(jnp.int32, (1, 128), 1)
    out_ref[0] = (jnp.where(li == 0, cls_s, 0.0)
                  + jnp.where(li == 1, bbox_s, 0.0)
                  + jnp.where(li == 2, ns_s, 0.0))


def _partials(cls_out, reg_out, target_boxes, interpret=False):
    Bn = cls_out.shape[0]
    tbT = jnp.transpose(target_boxes, (0, 2, 1))
    tbT8 = jnp.pad(tbT, ((0, 0), (0, 4), (0, 64)))
    return pl.pallas_call(
        _body,
        grid=(Bn,),
        in_specs=[
            pl.BlockSpec((1, 3, 128, 128), lambda b: (b, 0, 0, 0)),
            pl.BlockSpec((1, 12, 128, 128), lambda b: (b, 0, 0, 0)),
            pl.BlockSpec((1, 64, 4), lambda b: (b, 0, 0)),
            pl.BlockSpec((1, 8, 128), lambda b: (b, 0, 0)),
        ],
        out_specs=pl.BlockSpec((1, 1, 128), lambda b: (b, 0, 0)),
        out_shape=jax.ShapeDtypeStruct((Bn, 1, 128), jnp.float32),
        scratch_shapes=[
            pltpu.VMEM((3, 64, 128), jnp.float32),
            pltpu.VMEM((3, 128, 128), jnp.float32),
            pltpu.VMEM((5, 64, 128), jnp.float32),
        ],
        compiler_params=pltpu.CompilerParams(
            dimension_semantics=("parallel",)),
        interpret=interpret,
    )(cls_out, reg_out, target_boxes, tbT8)


def kernel(cls_out, reg_out, target_boxes):
    out = _partials(cls_out, reg_out, target_boxes)
    s = jnp.sum(out[:, 0, :3], axis=0)
    ns = jnp.maximum(s[2], 1.0)
    return jnp.stack([s[0] / ns, s[1] / ns])


# speedup vs baseline: 2.8778x; 2.8778x over previous
import math

import jax
import jax.numpy as jnp
from jax import lax
from jax.experimental import pallas as pl
from jax.experimental.pallas import tpu as pltpu

_HALVES = (16.0, 32.0, 64.0)
_IMG = 512.0
_THIRD = 1.0 / 3.0
_TWO7 = 2.0 / 7.0


def _body(cls_ref, reg_ref, tb_ref, tbT_ref, out_ref, wxq_scr, wyT_scr, sc_scr):
    f32 = jnp.float32
    tb = tb_ref[0]
    gx1 = tb[:, 0:1]
    gy1 = tb[:, 1:2]
    gx2 = tb[:, 2:3]
    gy2 = tb[:, 3:4]
    tbT = tbT_ref[0]
    gy1r = tbT[1:2, :]
    gy2r = tbT[3:4, :]

    lane = lax.broadcasted_iota(jnp.int32, (1, 128), 1).astype(f32)
    cxr = (lane + 0.5) * 4.0
    subl = lax.broadcasted_iota(jnp.int32, (128, 1), 0).astype(f32)
    cyc = (subl + 0.5) * 4.0

    area_g = (gx2 - gx1) * (gy2 - gy1)
    gt_best = jnp.full((64, 1), -1.0, f32)

    for si, h in enumerate(_HALVES):
        wx = jnp.clip(jnp.minimum(cxr + h, gx2) - jnp.maximum(cxr - h, gx1), 0.0)
        wyr = jnp.clip(jnp.minimum(cxr + h, gy2) - jnp.maximum(cxr - h, gy1), 0.0)
        C = area_g + (4.0 * h * h)
        wxq_scr[si] = wx * (1.0 / C)
        wyT = jnp.clip(jnp.minimum(cyc + h, gy2r) - jnp.maximum(cyc - h, gy1r), 0.0)
        wyT_scr[si] = wyT
        mx = jnp.max(wx, axis=1, keepdims=True)
        my = jnp.max(wyr, axis=1, keepdims=True)
        imax = mx * my
        iou = imax / jnp.maximum(C - imax, 1e-6)
        gt_best = jnp.maximum(gt_best, iou)

    t = gt_best - 1e-3
    thr_lowq = jnp.where(gt_best > 1e-3, t / (1.0 + t), 2.0)
    minthr = jnp.minimum(thr_lowq, _THIRD)
    ones_r = jnp.ones((1, 128), f32)
    sc_scr[0] = minthr * ones_r
    sc_scr[1] = ((gx1 + gx2) * 0.5) * ones_r
    sc_scr[2] = ((gy1 + gy2) * 0.5) * ones_r
    sc_scr[3] = jnp.log(jnp.maximum(gx2 - gx1, 1e-6)) * ones_r
    sc_scr[4] = jnp.log(jnp.maximum(gy2 - gy1, 1e-6)) * ones_r

    cls_s = jnp.float32(0.0)
    bbox_s = jnp.float32(0.0)
    ns_s = jnp.float32(0.0)

    TR = 32
    for si, h in enumerate(_HALVES):
        for ti in range(128 // TR):
            y0 = ti * TR
            wyT_t = wyT_scr[si, y0:y0 + TR, :]

            zero = jnp.zeros((TR, 128), f32)
            m = jnp.full((TR, 128), -1.0, f32)
            pm = jnp.full((TR, 128), -2.0, f32)
            mcx = zero; mcy = zero; mlw = zero; mlh = zero
            for g8 in range(8):
                gb = g8 * 8
                rows = wxq_scr[si, gb:gb + 8, :]
                thr8 = sc_scr[0, gb:gb + 8, :]
                cx8 = sc_scr[1, gb:gb + 8, :]
                cy8 = sc_scr[2, gb:gb + 8, :]
                lw8 = sc_scr[3, gb:gb + 8, :]
                lh8 = sc_scr[4, gb:gb + 8, :]
                for j in range(8):
                    q = wyT_t[:, gb + j:gb + j + 1] * rows[j:j + 1, :]
                    win = q > m
                    m = jnp.maximum(m, q)
                    pm = jnp.maximum(pm, q - thr8[j:j + 1, :])
                    mcx = jnp.where(win, cx8[j:j + 1, :], mcx)
                    mcy = jnp.where(win, cy8[j:j + 1, :], mcy)
                    mlw = jnp.where(win, lw8[j:j + 1, :], mlw)
                    mlh = jnp.where(win, lh8[j:j + 1, :], mlh)

            cyt = cyc[y0:y0 + TR]
            okx = (cxr >= h) & (cxr <= _IMG - h)
            oky = (cyt >= h) & (cyt <= _IMG - h)
            inside = okx & oky
            pos = (pm >= 0.0) & inside
            neg = (m < _TWO7) & (~pos) & inside
            posf = pos.astype(f32)
            lw = posf + neg.astype(f32)

            c = cls_ref[0, si, y0:y0 + TR, :]
            bce = jnp.maximum(c, 0.0) - c * posf + jnp.log1p(jnp.exp(-jnp.abs(c)))
            cls_s = cls_s + jnp.sum(bce * lw)

            aw = 2.0 * h
            dx = (mcx - cxr) / aw
            dy = (mcy - cyt) / aw
            dw = mlw - math.log(aw)
            dh = mlh - math.log(aw)
            r = reg_ref[0, 4 * si:4 * si + 4, y0:y0 + TR, :]
            l1 = (jnp.abs(r[0] - dx) + jnp.abs(r[1] - dy)
                  + jnp.abs(r[2] - dw) + jnp.abs(r[3] - dh))
            bbox_s = bbox_s + jnp.sum(l1 * posf)
            ns_s = ns_s + jnp.sum(lw)

    li = lax.broadcasted_iota(jnp.int32, (1, 128), 1)
    out_ref[0] = (jnp.where(li == 0, cls_s, 0.0)
                  + jnp.where(li == 1, bbox_s, 0.0)
                  + jnp.where(li == 2, ns_s, 0.0))


def _partials(cls_out, reg_out, target_boxes, interpret=False):
    Bn = cls_out.shape[0]
    tbT = jnp.transpose(target_boxes, (0, 2, 1))
    tbT8 = jnp.pad(tbT, ((0, 0), (0, 4), (0, 64)))
    return pl.pallas_call(
        _body,
        grid=(Bn,),
        in_specs=[
            pl.BlockSpec((1, 3, 128, 128), lambda b: (b, 0, 0, 0)),
            pl.BlockSpec((1, 12, 128, 128), lambda b: (b, 0, 0, 0)),
            pl.BlockSpec((1, 64, 4), lambda b: (b, 0, 0)),
            pl.BlockSpec((1, 8, 128), lambda b: (b, 0, 0)),
        ],
        out_specs=pl.BlockSpec((1, 1, 128), lambda b: (b, 0, 0)),
        out_shape=jax.ShapeDtypeStruct((Bn, 1, 128), jnp.float32),
        scratch_shapes=[
            pltpu.VMEM((3, 64, 128), jnp.float32),
            pltpu.VMEM((3, 128, 128), jnp.float32),
            pltpu.VMEM((5, 64, 128), jnp.float32),
        ],
        compiler_params=pltpu.CompilerParams(
            dimension_semantics=("parallel",)),
        interpret=interpret,
    )(cls_out, reg_out, target_boxes, tbT8)


def kernel(cls_out, reg_out, target_boxes):
    out = _partials(cls_out, reg_out, target_boxes)
    s = jnp.sum(out[:, 0, :3], axis=0)
    ns = jnp.maximum(s[2], 1.0)
    return jnp.stack([s[0] / ns, s[1] / ns])


# speedup vs baseline: 3.0931x; 1.0748x over previous
import math

import jax
import jax.numpy as jnp
from jax import lax
from jax.experimental import pallas as pl
from jax.experimental.pallas import tpu as pltpu

_HALVES = (16.0, 32.0, 64.0)
_IMG = 512.0
_THIRD = 1.0 / 3.0
_TWO7 = 2.0 / 7.0


def _body(cls_ref, reg_ref, tb_ref, tbT_ref, out_ref, wxq_scr, wyT_scr, sc_scr):
    f32 = jnp.float32
    tb = tb_ref[0]
    gx1 = tb[:, 0:1]
    gy1 = tb[:, 1:2]
    gx2 = tb[:, 2:3]
    gy2 = tb[:, 3:4]
    tbT = tbT_ref[0]
    gy1r = tbT[1:2, :]
    gy2r = tbT[3:4, :]

    lane = lax.broadcasted_iota(jnp.int32, (1, 128), 1).astype(f32)
    cxr = (lane + 0.5) * 4.0
    subl = lax.broadcasted_iota(jnp.int32, (128, 1), 0).astype(f32)
    cyc = (subl + 0.5) * 4.0

    area_g = (gx2 - gx1) * (gy2 - gy1)
    gt_best = jnp.full((64, 1), -1.0, f32)

    for si, h in enumerate(_HALVES):
        wx = jnp.clip(jnp.minimum(cxr + h, gx2) - jnp.maximum(cxr - h, gx1), 0.0)
        wyr = jnp.clip(jnp.minimum(cxr + h, gy2) - jnp.maximum(cxr - h, gy1), 0.0)
        C = area_g + (4.0 * h * h)
        wxq_scr[si] = wx * (1.0 / C)
        wyT = jnp.clip(jnp.minimum(cyc + h, gy2r) - jnp.maximum(cyc - h, gy1r), 0.0)
        wyT_scr[si] = wyT
        mx = jnp.max(wx, axis=1, keepdims=True)
        my = jnp.max(wyr, axis=1, keepdims=True)
        imax = mx * my
        iou = imax / jnp.maximum(C - imax, 1e-6)
        gt_best = jnp.maximum(gt_best, iou)

    t = gt_best - 1e-3
    thr_lowq = jnp.where(gt_best > 1e-3, t / (1.0 + t), 2.0)
    minthr = jnp.minimum(thr_lowq, _THIRD)
    ones_r = jnp.ones((1, 128), f32)
    sc_scr[0] = minthr * ones_r
    sc_scr[1] = ((gx1 + gx2) * 0.5) * ones_r
    sc_scr[2] = ((gy1 + gy2) * 0.5) * ones_r
    sc_scr[3] = jnp.log(jnp.maximum(gx2 - gx1, 1e-6)) * ones_r
    sc_scr[4] = jnp.log(jnp.maximum(gy2 - gy1, 1e-6)) * ones_r

    cls_s = jnp.float32(0.0)
    bbox_s = jnp.float32(0.0)
    ns_s = jnp.float32(0.0)

    TR = 32
    NT = 128 // TR
    for si, h in enumerate(_HALVES):
        st = {}
        for tp in range(NT // 2):
            tis = (2 * tp, 2 * tp + 1)
            wyt = {ti: wyT_scr[si, ti * TR:(ti + 1) * TR, :] for ti in tis}
            m = {ti: jnp.full((TR, 128), -1.0, f32) for ti in tis}
            pm = {ti: jnp.full((TR, 128), -2.0, f32) for ti in tis}
            mcx = {ti: jnp.zeros((TR, 128), f32) for ti in tis}
            mcy = {ti: jnp.zeros((TR, 128), f32) for ti in tis}
            mlw = {ti: jnp.zeros((TR, 128), f32) for ti in tis}
            mlh = {ti: jnp.zeros((TR, 128), f32) for ti in tis}
            for g8 in range(8):
                gb = g8 * 8
                rows = wxq_scr[si, gb:gb + 8, :]
                thr8 = sc_scr[0, gb:gb + 8, :]
                cx8 = sc_scr[1, gb:gb + 8, :]
                cy8 = sc_scr[2, gb:gb + 8, :]
                lw8 = sc_scr[3, gb:gb + 8, :]
                lh8 = sc_scr[4, gb:gb + 8, :]
                for j in range(8):
                    row = rows[j:j + 1, :]
                    thr = thr8[j:j + 1, :]
                    for ti in tis:
                        q = wyt[ti][:, gb + j:gb + j + 1] * row
                        win = q > m[ti]
                        m[ti] = jnp.maximum(m[ti], q)
                        pm[ti] = jnp.maximum(pm[ti], q - thr)
                        mcx[ti] = jnp.where(win, cx8[j:j + 1, :], mcx[ti])
                        mcy[ti] = jnp.where(win, cy8[j:j + 1, :], mcy[ti])
                        mlw[ti] = jnp.where(win, lw8[j:j + 1, :], mlw[ti])
                        mlh[ti] = jnp.where(win, lh8[j:j + 1, :], mlh[ti])
            for ti in tis:
                st[ti] = (m[ti], pm[ti], mcx[ti], mcy[ti], mlw[ti], mlh[ti])
        for ti in range(NT):
            y0 = ti * TR
            m, pm, mcx, mcy, mlw, mlh = st[ti]

            cyt = cyc[y0:y0 + TR]
            okx = (cxr >= h) & (cxr <= _IMG - h)
            oky = (cyt >= h) & (cyt <= _IMG - h)
            inside = okx & oky
            pos = (pm >= 0.0) & inside
            neg = (m < _TWO7) & (~pos) & inside
            posf = pos.astype(f32)
            lw = posf + neg.astype(f32)

            c = cls_ref[0, si, y0:y0 + TR, :]
            bce = jnp.maximum(c, 0.0) - c * posf + jnp.log1p(jnp.exp(-jnp.abs(c)))
            cls_s = cls_s + jnp.sum(bce * lw)

            aw = 2.0 * h
            dx = (mcx - cxr) / aw
            dy = (mcy - cyt) / aw
            dw = mlw - math.log(aw)
            dh = mlh - math.log(aw)
            r = reg_ref[0, 4 * si:4 * si + 4, y0:y0 + TR, :]
            l1 = (jnp.abs(r[0] - dx) + jnp.abs(r[1] - dy)
                  + jnp.abs(r[2] - dw) + jnp.abs(r[3] - dh))
            bbox_s = bbox_s + jnp.sum(l1 * posf)
            ns_s = ns_s + jnp.sum(lw)

    li = lax.broadcasted_iota(jnp.int32, (1, 128), 1)
    out_ref[0] = (jnp.where(li == 0, cls_s, 0.0)
                  + jnp.where(li == 1, bbox_s, 0.0)
                  + jnp.where(li == 2, ns_s, 0.0))


def _partials(cls_out, reg_out, target_boxes, interpret=False):
    Bn = cls_out.shape[0]
    tbT = jnp.transpose(target_boxes, (0, 2, 1))
    tbT8 = jnp.pad(tbT, ((0, 0), (0, 4), (0, 64)))
    return pl.pallas_call(
        _body,
        grid=(Bn,),
        in_specs=[
            pl.BlockSpec((1, 3, 128, 128), lambda b: (b, 0, 0, 0)),
            pl.BlockSpec((1, 12, 128, 128), lambda b: (b, 0, 0, 0)),
            pl.BlockSpec((1, 64, 4), lambda b: (b, 0, 0)),
            pl.BlockSpec((1, 8, 128), lambda b: (b, 0, 0)),
        ],
        out_specs=pl.BlockSpec((1, 1, 128), lambda b: (b, 0, 0)),
        out_shape=jax.ShapeDtypeStruct((Bn, 1, 128), jnp.float32),
        scratch_shapes=[
            pltpu.VMEM((3, 64, 128), jnp.float32),
            pltpu.VMEM((3, 128, 128), jnp.float32),
            pltpu.VMEM((5, 64, 128), jnp.float32),
        ],
        compiler_params=pltpu.CompilerParams(
            dimension_semantics=("parallel",)),
        interpret=interpret,
    )(cls_out, reg_out, target_boxes, tbT8)


def kernel(cls_out, reg_out, target_boxes):
    out = _partials(cls_out, reg_out, target_boxes)
    s = jnp.sum(out[:, 0, :3], axis=0)
    ns = jnp.maximum(s[2], 1.0)
    return jnp.stack([s[0] / ns, s[1] / ns])


# speedup vs baseline: 3.6354x; 1.1753x over previous
import math

import jax
import jax.numpy as jnp
from jax import lax
from jax.experimental import pallas as pl
from jax.experimental.pallas import tpu as pltpu

_HALVES = (16.0, 32.0, 64.0)
_IMG = 512.0
_THIRD = 1.0 / 3.0
_TWO7 = 2.0 / 7.0


def _body(cls_ref, reg_ref, tb_ref, tbT_ref, out_ref, wxq_scr, wyT_scr, sc_scr):
    f32 = jnp.float32
    tb = tb_ref[0]
    gx1 = tb[:, 0:1]
    gy1 = tb[:, 1:2]
    gx2 = tb[:, 2:3]
    gy2 = tb[:, 3:4]
    tbT = tbT_ref[0]
    gy1r = tbT[1:2, :]
    gy2r = tbT[3:4, :]

    lane = lax.broadcasted_iota(jnp.int32, (1, 128), 1).astype(f32)
    cxr = (lane + 0.5) * 4.0
    subl = lax.broadcasted_iota(jnp.int32, (128, 1), 0).astype(f32)
    cyc = (subl + 0.5) * 4.0

    area_g = (gx2 - gx1) * (gy2 - gy1)
    gt_best = jnp.full((64, 1), -1.0, f32)

    for si, h in enumerate(_HALVES):
        wx = jnp.clip(jnp.minimum(cxr + h, gx2) - jnp.maximum(cxr - h, gx1), 0.0)
        wyr = jnp.clip(jnp.minimum(cxr + h, gy2) - jnp.maximum(cxr - h, gy1), 0.0)
        C = area_g + (4.0 * h * h)
        wxq_scr[si] = wx * (1.0 / C)
        wyT = jnp.clip(jnp.minimum(cyc + h, gy2r) - jnp.maximum(cyc - h, gy1r), 0.0)
        wyT_scr[si] = wyT
        mx = jnp.max(wx, axis=1, keepdims=True)
        my = jnp.max(wyr, axis=1, keepdims=True)
        imax = mx * my
        iou = imax / jnp.maximum(C - imax, 1e-6)
        gt_best = jnp.maximum(gt_best, iou)

    t = gt_best - 1e-3
    thr_lowq = jnp.where(gt_best > 1e-3, t / (1.0 + t), 2.0)
    minthr = jnp.minimum(thr_lowq, _THIRD)
    ones_r = jnp.ones((1, 128), f32)
    sc_scr[0] = minthr * ones_r

    gx1r = tbT[0:1, :]
    gx2r = tbT[2:3, :]
    gcx_r = (gx1r + gx2r) * 0.5
    gcy_r = (gy1r + gy2r) * 0.5
    glw_r = jnp.log(jnp.maximum(gx2r - gx1r, 1e-6))
    glh_r = jnp.log(jnp.maximum(gy2r - gy1r, 1e-6))

    cls_a = jnp.zeros((32, 128), f32)
    bbox_a = jnp.zeros((32, 128), f32)
    ns_a = jnp.zeros((32, 128), f32)

    TR = 32
    NT = 128 // TR
    for si, h in enumerate(_HALVES):
        st = {}
        for tp in range(NT // 2):
            tis = (2 * tp, 2 * tp + 1)
            wyt = {ti: wyT_scr[si, ti * TR:(ti + 1) * TR, :] for ti in tis}
            mk = {ti: jnp.full((TR, 128), -1, jnp.int32) for ti in tis}
            pm = {ti: jnp.full((TR, 128), -2.0, f32) for ti in tis}
            for g8 in range(8):
                gb = g8 * 8
                rows = wxq_scr[si, gb:gb + 8, :]
                thr8 = sc_scr[0, gb:gb + 8, :]
                for j in range(8):
                    row = rows[j:j + 1, :]
                    thr = thr8[j:j + 1, :]
                    for ti in tis:
                        q = wyt[ti][:, gb + j:gb + j + 1] * row
                        kb = lax.bitcast_convert_type(q, jnp.int32)
                        key = (kb & ~63) | (63 - (gb + j))
                        mk[ti] = jnp.maximum(mk[ti], key)
                        pm[ti] = jnp.maximum(pm[ti], q - thr)
            for ti in tis:
                st[ti] = (mk[ti], pm[ti])
        for ti in range(NT):
            y0 = ti * TR
            mk, pm = st[ti]
            m = lax.bitcast_convert_type(mk & ~63, f32)
            ii = 63 - (mk & 63)
            mcx = jnp.take_along_axis(jnp.broadcast_to(gcx_r, (TR, 128)), ii, axis=1)
            mcy = jnp.take_along_axis(jnp.broadcast_to(gcy_r, (TR, 128)), ii, axis=1)
            mlw = jnp.take_along_axis(jnp.broadcast_to(glw_r, (TR, 128)), ii, axis=1)
            mlh = jnp.take_along_axis(jnp.broadcast_to(glh_r, (TR, 128)), ii, axis=1)

            cyt = cyc[y0:y0 + TR]
            okx = (cxr >= h) & (cxr <= _IMG - h)
            oky = (cyt >= h) & (cyt <= _IMG - h)
            inside = okx & oky
            pos = (pm >= 0.0) & inside
            neg = (m < _TWO7) & (~pos) & inside
            posf = pos.astype(f32)
            lw = posf + neg.astype(f32)

            c = cls_ref[0, si, y0:y0 + TR, :]
            bce = jnp.maximum(c, 0.0) - c * posf + jnp.log1p(jnp.exp(-jnp.abs(c)))
            cls_a = cls_a + bce * lw

            aw = 2.0 * h
            dx = (mcx - cxr) / aw
            dy = (mcy - cyt) / aw
            dw = mlw - math.log(aw)
            dh = mlh - math.log(aw)
            r = reg_ref[0, 4 * si:4 * si + 4, y0:y0 + TR, :]
            l1 = (jnp.abs(r[0] - dx) + jnp.abs(r[1] - dy)
                  + jnp.abs(r[2] - dw) + jnp.abs(r[3] - dh))
            bbox_a = bbox_a + l1 * posf
            ns_a = ns_a + lw

    cls_s = jnp.sum(cls_a)
    bbox_s = jnp.sum(bbox_a)
    ns_s = jnp.sum(ns_a)
    li = lax.broadcasted_iota(jnp.int32, (1, 128), 1)
    out_ref[0] = (jnp.where(li == 0, cls_s, 0.0)
                  + jnp.where(li == 1, bbox_s, 0.0)
                  + jnp.where(li == 2, ns_s, 0.0))


def _partials(cls_out, reg_out, target_boxes, interpret=False):
    Bn = cls_out.shape[0]
    tbT = jnp.transpose(target_boxes, (0, 2, 1))
    tbT8 = jnp.pad(tbT, ((0, 0), (0, 4), (0, 64)))
    return pl.pallas_call(
        _body,
        grid=(Bn,),
        in_specs=[
            pl.BlockSpec((1, 3, 128, 128), lambda b: (b, 0, 0, 0)),
            pl.BlockSpec((1, 12, 128, 128), lambda b: (b, 0, 0, 0)),
            pl.BlockSpec((1, 64, 4), lambda b: (b, 0, 0)),
            pl.BlockSpec((1, 8, 128), lambda b: (b, 0, 0)),
        ],
        out_specs=pl.BlockSpec((1, 1, 128), lambda b: (b, 0, 0)),
        out_shape=jax.ShapeDtypeStruct((Bn, 1, 128), jnp.float32),
        scratch_shapes=[
            pltpu.VMEM((3, 64, 128), jnp.float32),
            pltpu.VMEM((3, 128, 128), jnp.float32),
            pltpu.VMEM((1, 64, 128), jnp.float32),
        ],
        compiler_params=pltpu.CompilerParams(
            dimension_semantics=("parallel",)),
        interpret=interpret,
    )(cls_out, reg_out, target_boxes, tbT8)


def kernel(cls_out, reg_out, target_boxes):
    out = _partials(cls_out, reg_out, target_boxes)
    s = jnp.sum(out[:, 0, :3], axis=0)
    ns = jnp.maximum(s[2], 1.0)
    return jnp.stack([s[0] / ns, s[1] / ns])


# speedup vs baseline: 3.7203x; 1.0234x over previous
import math

import jax
import jax.numpy as jnp
from jax import lax
from jax.experimental import pallas as pl
from jax.experimental.pallas import tpu as pltpu

_HALVES = (16.0, 32.0, 64.0)
_IMG = 512.0
_THIRD = 1.0 / 3.0
_TWO7 = 2.0 / 7.0


def _body(cls_ref, reg_ref, tb_ref, tbT_ref, out_ref, wxq_scr, wyT_scr, sc_scr):
    f32 = jnp.float32
    tb = tb_ref[0]
    gx1 = tb[:, 0:1]
    gy1 = tb[:, 1:2]
    gx2 = tb[:, 2:3]
    gy2 = tb[:, 3:4]
    tbT = tbT_ref[0]
    gy1r = tbT[1:2, :]
    gy2r = tbT[3:4, :]

    lane = lax.broadcasted_iota(jnp.int32, (1, 128), 1).astype(f32)
    cxr = (lane + 0.5) * 4.0
    subl = lax.broadcasted_iota(jnp.int32, (128, 1), 0).astype(f32)
    cyc = (subl + 0.5) * 4.0

    area_g = (gx2 - gx1) * (gy2 - gy1)
    gt_best = jnp.full((64, 1), -1.0, f32)

    for si, h in enumerate(_HALVES):
        wx = jnp.clip(jnp.minimum(cxr + h, gx2) - jnp.maximum(cxr - h, gx1), 0.0)
        wyr = jnp.clip(jnp.minimum(cxr + h, gy2) - jnp.maximum(cxr - h, gy1), 0.0)
        C = area_g + (4.0 * h * h)
        wxq_scr[si] = wx * (1.0 / C)
        wyT = jnp.clip(jnp.minimum(cyc + h, gy2r) - jnp.maximum(cyc - h, gy1r), 0.0)
        wyT_scr[si] = wyT
        mx = jnp.max(wx, axis=1, keepdims=True)
        my = jnp.max(wyr, axis=1, keepdims=True)
        imax = mx * my
        iou = imax / jnp.maximum(C - imax, 1e-6)
        gt_best = jnp.maximum(gt_best, iou)

    t = gt_best - 1e-3
    thr_lowq = jnp.where(gt_best > 1e-3, t / (1.0 + t), 2.0)
    minthr = jnp.minimum(thr_lowq, _THIRD)
    ones_r = jnp.ones((1, 128), f32)
    sc_scr[0] = minthr * ones_r

    gx1r = tbT[0:1, :]
    gx2r = tbT[2:3, :]
    gcx_r = (gx1r + gx2r) * 0.5
    gcy_r = (gy1r + gy2r) * 0.5
    glw_r = jnp.log(jnp.maximum(gx2r - gx1r, 1e-6))
    glh_r = jnp.log(jnp.maximum(gy2r - gy1r, 1e-6))

    cls_a = jnp.zeros((32, 128), f32)
    bbox_a = jnp.zeros((32, 128), f32)
    ns_a = jnp.zeros((32, 128), f32)

    TR = 32
    NT = 128 // TR
    for si, h in enumerate(_HALVES):
        tis = tuple(range(NT))
        wyt = {ti: wyT_scr[si, ti * TR:(ti + 1) * TR, :] for ti in tis}
        mkf = {ti: jnp.full((TR, 128), -1.0, f32) for ti in tis}
        pm = {ti: jnp.full((TR, 128), -2.0, f32) for ti in tis}
        for g8 in range(8):
            gb = g8 * 8
            rows = wxq_scr[si, gb:gb + 8, :]
            thr8 = sc_scr[0, gb:gb + 8, :]
            for j in range(8):
                row = rows[j:j + 1, :]
                thr = thr8[j:j + 1, :]
                for ti in tis:
                    q = wyt[ti][:, gb + j:gb + j + 1] * row
                    kb = lax.bitcast_convert_type(q, jnp.int32)
                    keyf = lax.bitcast_convert_type(
                        (kb & ~63) | (63 - (gb + j)), f32)
                    mkf[ti] = jnp.maximum(mkf[ti], keyf)
                    pm[ti] = jnp.maximum(pm[ti], q - thr)
        for ti in range(NT):
            y0 = ti * TR
            mk = lax.bitcast_convert_type(mkf[ti], jnp.int32)
            pmv = pm[ti]
            m = lax.bitcast_convert_type(mk & ~63, f32)
            ii = 63 - (mk & 63)
            mcx = jnp.take_along_axis(jnp.broadcast_to(gcx_r, (TR, 128)), ii, axis=1)
            mcy = jnp.take_along_axis(jnp.broadcast_to(gcy_r, (TR, 128)), ii, axis=1)
            mlw = jnp.take_along_axis(jnp.broadcast_to(glw_r, (TR, 128)), ii, axis=1)
            mlh = jnp.take_along_axis(jnp.broadcast_to(glh_r, (TR, 128)), ii, axis=1)

            cyt = cyc[y0:y0 + TR]
            okx = (cxr >= h) & (cxr <= _IMG - h)
            oky = (cyt >= h) & (cyt <= _IMG - h)
            inside = okx & oky
            pos = (pmv >= 0.0) & inside
            neg = (m < _TWO7) & (~pos) & inside
            posf = pos.astype(f32)
            lw = posf + neg.astype(f32)

            c = cls_ref[0, si, y0:y0 + TR, :]
            bce = jnp.maximum(c, 0.0) - c * posf + jnp.log1p(jnp.exp(-jnp.abs(c)))
            cls_a = cls_a + bce * lw

            aw = 2.0 * h
            dx = (mcx - cxr) / aw
            dy = (mcy - cyt) / aw
            dw = mlw - math.log(aw)
            dh = mlh - math.log(aw)
            r = reg_ref[0, 4 * si:4 * si + 4, y0:y0 + TR, :]
            l1 = (jnp.abs(r[0] - dx) + jnp.abs(r[1] - dy)
                  + jnp.abs(r[2] - dw) + jnp.abs(r[3] - dh))
            bbox_a = bbox_a + l1 * posf
            ns_a = ns_a + lw

    cls_s = jnp.sum(cls_a)
    bbox_s = jnp.sum(bbox_a)
    ns_s = jnp.sum(ns_a)
    li = lax.broadcasted_iota(jnp.int32, (1, 128), 1)
    out_ref[0] = (jnp.where(li == 0, cls_s, 0.0)
                  + jnp.where(li == 1, bbox_s, 0.0)
                  + jnp.where(li == 2, ns_s, 0.0))


def _partials(cls_out, reg_out, target_boxes, interpret=False):
    Bn = cls_out.shape[0]
    tbT = jnp.transpose(target_boxes, (0, 2, 1))
    tbT8 = jnp.pad(tbT, ((0, 0), (0, 4), (0, 64)))
    return pl.pallas_call(
        _body,
        grid=(Bn,),
        in_specs=[
            pl.BlockSpec((1, 3, 128, 128), lambda b: (b, 0, 0, 0)),
            pl.BlockSpec((1, 12, 128, 128), lambda b: (b, 0, 0, 0)),
            pl.BlockSpec((1, 64, 4), lambda b: (b, 0, 0)),
            pl.BlockSpec((1, 8, 128), lambda b: (b, 0, 0)),
        ],
        out_specs=pl.BlockSpec((1, 1, 128), lambda b: (b, 0, 0)),
        out_shape=jax.ShapeDtypeStruct((Bn, 1, 128), jnp.float32),
        scratch_shapes=[
            pltpu.VMEM((3, 64, 128), jnp.float32),
            pltpu.VMEM((3, 128, 128), jnp.float32),
            pltpu.VMEM((1, 64, 128), jnp.float32),
        ],
        compiler_params=pltpu.CompilerParams(
            dimension_semantics=("parallel",)),
        interpret=interpret,
    )(cls_out, reg_out, target_boxes, tbT8)


def kernel(cls_out, reg_out, target_boxes):
    out = _partials(cls_out, reg_out, target_boxes)
    s = jnp.sum(out[:, 0, :3], axis=0)
    ns = jnp.maximum(s[2], 1.0)
    return jnp.stack([s[0] / ns, s[1] / ns])
